# Initial kernel scaffold; baseline (speedup 1.0000x reference)
import jax
import jax.numpy as jnp
from jax.experimental import pallas as pl


def kernel(embedded_features, targets, hsa):
    raise NotImplementedError("write your pallas kernel here")


# speedup vs baseline: 1.5885x; 1.5885x over previous
import functools

import jax
import jax.numpy as jnp
from jax import lax
from jax.experimental import pallas as pl
from jax.experimental.pallas import tpu as pltpu
from jax.experimental.pallas import tpu_sc as plsc


def kernel(embedded_features, targets, hsa):
    B, D = embedded_features.shape
    C = hsa.shape[0]
    info = plsc.get_sparse_core_info()
    NC, NS, L = info.num_cores, info.num_subcores, info.num_lanes
    NW = NC * NS
    b_per_w = B // NW
    groups = b_per_w // L

    mesh = plsc.VectorSubcoreMesh(core_axis_name="c", subcore_axis_name="s")

    @functools.partial(
        pl.kernel,
        mesh=mesh,
        compiler_params=pltpu.CompilerParams(needs_layout_passes=False),
        out_type=jax.ShapeDtypeStruct((NW, L), jnp.float32),
        scratch_types=[
            pltpu.VMEM((b_per_w,), jnp.int32),
            pltpu.VMEM((b_per_w * D,), jnp.float32),
            pltpu.VMEM((C * D,), jnp.float32),
            pltpu.VMEM((L,), jnp.float32),
        ],
    )
    def run(feat_hbm, tgt_hbm, hsa_hbm, out_hbm, idx_v, feat_v, tbl_v, acc_v):
        wid = lax.axis_index("s") * NC + lax.axis_index("c")
        base = wid * b_per_w
        pltpu.sync_copy(hsa_hbm, tbl_v)
        pltpu.sync_copy(tgt_hbm.at[pl.ds(base, b_per_w)], idx_v)
        pltpu.sync_copy(feat_hbm.at[pl.ds(base * D, b_per_w * D)], feat_v)

        lane = lax.iota(jnp.int32, L)

        def body(g, acc):
            idx_vec = idx_v[pl.ds(g * L, L)] * D
            row_base = (lane + g * L) * D
            for d in range(D):
                tcol = plsc.load_gather(tbl_v, [idx_vec + d])
                fcol = plsc.load_gather(feat_v, [row_base + d])
                diff = fcol - tcol
                acc = acc + diff * diff
            return acc

        acc = lax.fori_loop(0, groups, body, jnp.zeros((L,), jnp.float32))
        acc_v[...] = acc
        pltpu.sync_copy(acc_v, out_hbm.at[wid])

    partials = run(
        embedded_features.reshape(-1), targets.astype(jnp.int32), hsa.reshape(-1)
    )
    return jnp.sum(partials) / B


# speedup vs baseline: 1.6103x; 1.0137x over previous
import functools

import jax
import jax.numpy as jnp
from jax import lax
from jax.experimental import pallas as pl
from jax.experimental.pallas import tpu as pltpu
from jax.experimental.pallas import tpu_sc as plsc


def kernel(embedded_features, targets, hsa):
    B, D = embedded_features.shape
    C = hsa.shape[0]
    info = plsc.get_sparse_core_info()
    NC, NS, L = info.num_cores, info.num_subcores, info.num_lanes
    NW = NC * NS
    b_per_w = B // NW
    groups = b_per_w // L

    mesh = plsc.VectorSubcoreMesh(core_axis_name="c", subcore_axis_name="s")

    @functools.partial(
        pl.kernel,
        mesh=mesh,
        compiler_params=pltpu.CompilerParams(needs_layout_passes=False),
        out_type=jax.ShapeDtypeStruct((NW, L), jnp.float32),
        scratch_types=[
            pltpu.VMEM((b_per_w,), jnp.int32),
            pltpu.VMEM((b_per_w * D,), jnp.float32),
            pltpu.VMEM((C * D,), jnp.float32),
            pltpu.VMEM((L,), jnp.float32),
        ],
    )
    def run(feat_hbm, tgt_hbm, hsa_hbm, out_hbm, idx_v, feat_v, tbl_v, acc_v):
        wid = lax.axis_index("s") * NC + lax.axis_index("c")
        base = wid * b_per_w
        pltpu.sync_copy(hsa_hbm, tbl_v)
        pltpu.sync_copy(tgt_hbm.at[pl.ds(base, b_per_w)], idx_v)
        pltpu.sync_copy(feat_hbm.at[pl.ds(base * D, b_per_w * D)], feat_v)

        lane_d = lax.iota(jnp.int32, L) * D
        zero = jnp.zeros((L,), jnp.float32)
        n_acc = 8

        @plsc.parallel_loop(0, groups, unroll=2, carry=(zero,) * n_acc)
        def accs(g, accs):
            accs = list(accs)
            idx16 = idx_v[pl.ds(g * L, L)] * D
            row_base = lane_d + g * (L * D)
            for d in range(D):
                tcol = plsc.load_gather(tbl_v, [idx16 + d])
                fcol = plsc.load_gather(feat_v, [row_base + d])
                diff = fcol - tcol
                accs[d % n_acc] = accs[d % n_acc] + diff * diff
            return tuple(accs)

        acc = zero
        for a in accs:
            acc = acc + a
        acc_v[...] = acc
        pltpu.sync_copy(acc_v, out_hbm.at[wid])

    partials = run(
        embedded_features.reshape(-1), targets.astype(jnp.int32), hsa.reshape(-1)
    )
    return jnp.sum(partials) / B


# speedup vs baseline: 1.8256x; 1.1337x over previous
import functools

import jax
import jax.numpy as jnp
from jax import lax
from jax.experimental import pallas as pl
from jax.experimental.pallas import tpu as pltpu
from jax.experimental.pallas import tpu_sc as plsc


def kernel(embedded_features, targets, hsa):
    B, D = embedded_features.shape
    C = hsa.shape[0]
    info = plsc.get_sparse_core_info()
    NC, NS, L = info.num_cores, info.num_subcores, info.num_lanes
    NW = NC * NS
    b_per_w = B // NW

    mesh = plsc.VectorSubcoreMesh(core_axis_name="c", subcore_axis_name="s")

    @functools.partial(
        pl.kernel,
        mesh=mesh,
        compiler_params=pltpu.CompilerParams(needs_layout_passes=False),
        out_type=jax.ShapeDtypeStruct((NW, L), jnp.float32),
        scratch_types=[
            pltpu.VMEM((b_per_w,), jnp.int32),
            pltpu.VMEM((b_per_w, D), jnp.float32),
            pltpu.VMEM((C, D), jnp.float32),
            pltpu.VMEM((L,), jnp.float32),
        ],
    )
    def run(feat_hbm, tgt_hbm, hsa_hbm, out_hbm, idx_v, feat_v, tbl_v, acc_v):
        wid = lax.axis_index("s") * NC + lax.axis_index("c")
        base = wid * b_per_w
        pltpu.sync_copy(hsa_hbm, tbl_v)
        pltpu.sync_copy(tgt_hbm.at[pl.ds(base, b_per_w)], idx_v)
        pltpu.sync_copy(feat_hbm.at[pl.ds(base, b_per_w)], feat_v)

        zero = jnp.zeros((L,), jnp.float32)
        n_acc = 8

        @plsc.parallel_loop(0, b_per_w, step=L, carry=(zero,) * n_acc)
        def accs(k, accs):
            accs = list(accs)
            idx_vec = idx_v[pl.ds(k, L)]
            for j in range(L):
                t = idx_vec[j]
                diff = feat_v[k + j] - tbl_v[t]
                accs[j % n_acc] = accs[j % n_acc] + diff * diff
            return tuple(accs)

        acc = zero
        for a in accs:
            acc = acc + a
        acc_v[...] = acc
        pltpu.sync_copy(acc_v, out_hbm.at[wid])

    partials = run(embedded_features, targets.astype(jnp.int32), hsa)
    return jnp.sum(partials) / B
